# baseline (speedup 1.0000x reference)
import functools

import jax
import jax.numpy as jnp
from jax import lax
from jax.experimental import pallas as pl
from jax.experimental.pallas import tpu as pltpu
from jax.experimental.pallas import tpu_sc as plsc

NC = 2
NS = 16
NW = NC * NS
CHUNK = 128
LANES = 16


GRP = 8


def _make_spmm(n, h, ngrp):
    zbuf_rows = 64
    zrows_tile = -(-(-(-(n + 1) // NS)) // zbuf_rows) * zbuf_rows
    acc_rows = NS * zrows_tile
    mesh = plsc.VectorSubcoreMesh(core_axis_name="c", subcore_axis_name="s")

    @functools.partial(
        pl.kernel,
        mesh=mesh,
        compiler_params=pltpu.CompilerParams(use_tc_tiling_on_sc=False),
        out_type=jax.ShapeDtypeStruct((NC, acc_rows, h), jnp.float32),
        scratch_types=[
            pltpu.VMEM((ngrp, GRP, CHUNK), jnp.int32),
            pltpu.VMEM((ngrp, GRP, CHUNK), jnp.int32),
            pltpu.VMEM((GRP, CHUNK, h), jnp.float32),
            pltpu.VMEM((GRP, CHUNK, h), jnp.float32),
            pltpu.VMEM((zbuf_rows, h), jnp.float32),
            pltpu.VMEM_SHARED((acc_rows, h), jnp.float32),
            pltpu.SemaphoreType.DMA,
            pltpu.SemaphoreType.DMA,
            pltpu.SemaphoreType.DMA,
            pltpu.SemaphoreType.DMA,
        ],
    )
    def spmm(src_hbm, dst_hbm, tbl_hbm, out_hbm, srcv, dstv, rows0, rows1,
             zbuf, acc, gsem0, gsem1, ssem0, ssem1):
        cid = lax.axis_index("c")
        sid = lax.axis_index("s")
        wid = cid * NS + sid

        zero = jnp.zeros((LANES,), jnp.float32)

        def zrow(r, carry):
            for j in range(h // LANES):
                zbuf[r, pl.ds(j * LANES, LANES)] = zero
            return carry

        lax.fori_loop(0, zbuf_rows, zrow, 0)
        for t in range(zrows_tile // zbuf_rows):
            pltpu.sync_copy(
                zbuf, acc.at[pl.ds(sid * zrows_tile + t * zbuf_rows, zbuf_rows)])
        plsc.subcore_barrier()

        pltpu.sync_copy(src_hbm.at[wid], srcv)
        pltpu.sync_copy(dst_hbm.at[wid], dstv)

        def fire_gathers(s, buf, sem):
            for k in range(GRP):
                pltpu.async_copy(tbl_hbm.at[srcv.at[s, k]], buf.at[k], sem)

        def drain_gathers(buf, sem):
            for k in range(GRP):
                pltpu.make_async_copy(
                    tbl_hbm.at[srcv.at[0, k]], buf.at[k], sem).wait()

        def fire_scatters(s, buf, sem):
            for k in range(GRP):
                pltpu.async_copy(buf.at[k], acc.at[dstv.at[s, k]], sem,
                                 add=True)

        def drain_scatters(s, buf, sem):
            for k in range(GRP):
                pltpu.make_async_copy(buf.at[k], acc.at[dstv.at[s, k]],
                                      sem).wait()

        banks = ((rows0, gsem0, ssem0), (rows1, gsem1, ssem1))

        def step(s, carry):
            def run(p):
                buf, gsem, ssem = banks[p]
                obuf, ogsem, ossem = banks[1 - p]

                @pl.when(s >= 1)
                def _():
                    drain_scatters(s - 1, obuf, ossem)

                @pl.when(s + 1 < ngrp)
                def _():
                    fire_gathers(s + 1, obuf, ogsem)

                drain_gathers(buf, gsem)
                fire_scatters(s, buf, ssem)

            @pl.when(lax.rem(s, 2) == 0)
            def _():
                run(0)

            @pl.when(lax.rem(s, 2) == 1)
            def _():
                run(1)

            return carry

        fire_gathers(0, rows0, gsem0)
        lax.fori_loop(0, ngrp, step, 0)
        lbuf, _, lssem = banks[(ngrp - 1) % 2]
        drain_scatters(ngrp - 1, lbuf, lssem)
        plsc.subcore_barrier()

        pltpu.sync_copy(acc.at[pl.ds(sid * zrows_tile, zrows_tile)],
                        out_hbm.at[cid, pl.ds(sid * zrows_tile, zrows_tile)])

    return spmm


def _spmm_partials(edge_index, table):
    n, h = table.shape
    e = edge_index.shape[1]
    gsz = GRP * CHUNK
    per_w = -(-e // (NW * 2 * gsz)) * (2 * gsz)
    e_pad = per_w * NW
    src = jnp.concatenate(
        [edge_index[0], jnp.zeros((e_pad - e,), jnp.int32)]).reshape(
            NW, per_w // gsz, GRP, CHUNK)
    dst = jnp.concatenate(
        [edge_index[1], jnp.full((e_pad - e,), n, jnp.int32)]).reshape(
            NW, per_w // gsz, GRP, CHUNK)
    return _make_spmm(n, h, per_w // gsz)(src, dst, table)


def _mm1_body(x_ref, w_ref, o_ref):
    o_ref[...] = jnp.dot(x_ref[...], w_ref[...],
                         preferred_element_type=jnp.float32,
                         precision=lax.Precision.HIGHEST)


def _mm1(x, w, bn):
    n, d = x.shape
    _, h = w.shape
    return pl.pallas_call(
        _mm1_body,
        grid=(n // bn,),
        in_specs=[pl.BlockSpec((bn, d), lambda i: (i, 0)),
                  pl.BlockSpec((d, h), lambda i: (0, 0))],
        out_specs=pl.BlockSpec((bn, h), lambda i: (i, 0)),
        out_shape=jax.ShapeDtypeStruct((n, h), jnp.float32),
    )(x, w)


def _relu_mm_body(p0_ref, p1_ref, w_ref, o_ref):
    hblk = jnp.maximum(p0_ref[0] + p1_ref[0], 0.0)
    o_ref[...] = jnp.dot(hblk, w_ref[...],
                         preferred_element_type=jnp.float32,
                         precision=lax.Precision.HIGHEST)


def _relu_mm(p, w, n, bn):
    _, _, h1 = p.shape
    _, h2 = w.shape
    return pl.pallas_call(
        _relu_mm_body,
        grid=(n // bn,),
        in_specs=[pl.BlockSpec((1, bn, h1), lambda i: (0, i, 0)),
                  pl.BlockSpec((1, bn, h1), lambda i: (1, i, 0)),
                  pl.BlockSpec((h1, h2), lambda i: (0, 0))],
        out_specs=pl.BlockSpec((bn, h2), lambda i: (i, 0)),
        out_shape=jax.ShapeDtypeStruct((n, h2), jnp.float32),
    )(p, p, w)


def _decoder_body(pi0_ref, pi1_ref, pj0_ref, pj1_ref, z_ref, r_ref):
    zi = pi0_ref[0] + pi1_ref[0]
    zj = pj0_ref[0] + pj1_ref[0]
    z_ref[...] = zi
    r_ref[...] = lax.dot_general(
        zi, zj, dimension_numbers=(((1,), (1,)), ((), ())),
        preferred_element_type=jnp.float32)


def _decoder(p, n, bn):
    _, _, h = p.shape
    return pl.pallas_call(
        _decoder_body,
        grid=(n // bn,),
        in_specs=[pl.BlockSpec((1, bn, h), lambda i: (0, i, 0)),
                  pl.BlockSpec((1, bn, h), lambda i: (1, i, 0)),
                  pl.BlockSpec((1, n, h), lambda i: (0, 0, 0)),
                  pl.BlockSpec((1, n, h), lambda i: (1, 0, 0))],
        out_specs=[pl.BlockSpec((bn, h), lambda i: (i, 0)),
                   pl.BlockSpec((bn, n), lambda i: (i, 0))],
        out_shape=[jax.ShapeDtypeStruct((n, h), jnp.float32),
                   jax.ShapeDtypeStruct((n, n), jnp.float32)],
    )(p, p, p, p)


def kernel(x, edge_index, W1, W2):
    n = x.shape[0]
    h = _mm1(x, W1, 1000)
    p1 = _spmm_partials(edge_index, h)
    h2 = _relu_mm(p1, W2, n, 1000)
    p2 = _spmm_partials(edge_index, h2)
    z, recon = _decoder(p2, n, 400)
    return (z, jnp.reshape(recon, (-1,)))


# speedup vs baseline: 1.3104x; 1.3104x over previous
import functools

import jax
import jax.numpy as jnp
from jax import lax
from jax.experimental import pallas as pl
from jax.experimental.pallas import tpu as pltpu
from jax.experimental.pallas import tpu_sc as plsc

NC = 2
NS = 16
NW = NC * NS
CHUNK = 128
LANES = 16


GRP = 4


def _make_spmm(n, h, ngrp):
    zbuf_rows = 64
    zrows_tile = -(-(-(-(n + 1) // NS)) // zbuf_rows) * zbuf_rows
    acc_rows = NS * zrows_tile
    mesh = plsc.VectorSubcoreMesh(core_axis_name="c", subcore_axis_name="s")

    @functools.partial(
        pl.kernel,
        mesh=mesh,
        compiler_params=pltpu.CompilerParams(use_tc_tiling_on_sc=False),
        out_type=jax.ShapeDtypeStruct((NC, acc_rows, h), jnp.float32),
        scratch_types=[
            pltpu.VMEM((ngrp, GRP, CHUNK), jnp.int32),
            pltpu.VMEM((ngrp, GRP, CHUNK), jnp.int32),
            pltpu.VMEM((GRP, CHUNK, h), jnp.float32),
            pltpu.VMEM((GRP, CHUNK, h), jnp.float32),
            pltpu.VMEM((zbuf_rows, h), jnp.float32),
            pltpu.VMEM_SHARED((acc_rows, h), jnp.float32),
            pltpu.VMEM_SHARED((n, h), jnp.float32),
            pltpu.SemaphoreType.DMA,
            pltpu.SemaphoreType.DMA,
            pltpu.SemaphoreType.DMA,
            pltpu.SemaphoreType.DMA,
        ],
    )
    def spmm(src_hbm, dst_hbm, tbl_hbm, out_hbm, srcv, dstv, rows0, rows1,
             zbuf, acc, tbl_s, gsem0, gsem1, ssem0, ssem1):
        cid = lax.axis_index("c")
        sid = lax.axis_index("s")
        wid = cid * NS + sid

        zero = jnp.zeros((LANES,), jnp.float32)

        def zrow(r, carry):
            for j in range(h // LANES):
                zbuf[r, pl.ds(j * LANES, LANES)] = zero
            return carry

        lax.fori_loop(0, zbuf_rows, zrow, 0)
        for t in range(zrows_tile // zbuf_rows):
            pltpu.sync_copy(
                zbuf, acc.at[pl.ds(sid * zrows_tile + t * zbuf_rows, zbuf_rows)])
        plsc.subcore_barrier()

        pltpu.sync_copy(src_hbm.at[wid], srcv)
        pltpu.sync_copy(dst_hbm.at[wid], dstv)
        trows = n // NS
        tbase = sid * trows
        pltpu.sync_copy(tbl_hbm.at[pl.ds(tbase, trows)],
                        tbl_s.at[pl.ds(tbase, trows)])
        plsc.subcore_barrier()

        def fire_gathers(s, buf, sem):
            for k in range(GRP):
                pltpu.async_copy(tbl_s.at[srcv.at[s, k]], buf.at[k], sem)

        def drain_gathers(buf, sem):
            for k in range(GRP):
                pltpu.make_async_copy(
                    tbl_s.at[srcv.at[0, k]], buf.at[k], sem).wait()

        def fire_scatters(s, buf, sem):
            for k in range(GRP):
                pltpu.async_copy(buf.at[k], acc.at[dstv.at[s, k]], sem,
                                 add=True)

        def drain_scatters(s, buf, sem):
            for k in range(GRP):
                pltpu.make_async_copy(buf.at[k], acc.at[dstv.at[s, k]],
                                      sem).wait()

        banks = ((rows0, gsem0, ssem0), (rows1, gsem1, ssem1))

        def step(s, carry):
            def run(p):
                buf, gsem, ssem = banks[p]
                obuf, ogsem, ossem = banks[1 - p]

                @pl.when(s >= 1)
                def _():
                    drain_scatters(s - 1, obuf, ossem)

                @pl.when(s + 1 < ngrp)
                def _():
                    fire_gathers(s + 1, obuf, ogsem)

                drain_gathers(buf, gsem)
                fire_scatters(s, buf, ssem)

            @pl.when(lax.rem(s, 2) == 0)
            def _():
                run(0)

            @pl.when(lax.rem(s, 2) == 1)
            def _():
                run(1)

            return carry

        fire_gathers(0, rows0, gsem0)
        lax.fori_loop(0, ngrp, step, 0)
        lbuf, _, lssem = banks[(ngrp - 1) % 2]
        drain_scatters(ngrp - 1, lbuf, lssem)
        plsc.subcore_barrier()

        pltpu.sync_copy(acc.at[pl.ds(sid * zrows_tile, zrows_tile)],
                        out_hbm.at[cid, pl.ds(sid * zrows_tile, zrows_tile)])

    return spmm


def _spmm_partials(edge_index, table):
    n, h = table.shape
    e = edge_index.shape[1]
    gsz = GRP * CHUNK
    per_w = -(-e // (NW * 2 * gsz)) * (2 * gsz)
    e_pad = per_w * NW
    src = jnp.concatenate(
        [edge_index[0], jnp.zeros((e_pad - e,), jnp.int32)]).reshape(
            NW, per_w // gsz, GRP, CHUNK)
    dst = jnp.concatenate(
        [edge_index[1], jnp.full((e_pad - e,), n, jnp.int32)]).reshape(
            NW, per_w // gsz, GRP, CHUNK)
    return _make_spmm(n, h, per_w // gsz)(src, dst, table)


def _mm1_body(x_ref, w_ref, o_ref):
    o_ref[...] = jnp.dot(x_ref[...], w_ref[...],
                         preferred_element_type=jnp.float32,
                         precision=lax.Precision.HIGHEST)


def _mm1(x, w, bn):
    n, d = x.shape
    _, h = w.shape
    return pl.pallas_call(
        _mm1_body,
        grid=(n // bn,),
        in_specs=[pl.BlockSpec((bn, d), lambda i: (i, 0)),
                  pl.BlockSpec((d, h), lambda i: (0, 0))],
        out_specs=pl.BlockSpec((bn, h), lambda i: (i, 0)),
        out_shape=jax.ShapeDtypeStruct((n, h), jnp.float32),
    )(x, w)


def _relu_mm_body(p0_ref, p1_ref, w_ref, o_ref):
    hblk = jnp.maximum(p0_ref[0] + p1_ref[0], 0.0)
    o_ref[...] = jnp.dot(hblk, w_ref[...],
                         preferred_element_type=jnp.float32,
                         precision=lax.Precision.HIGHEST)


def _relu_mm(p, w, n, bn):
    _, _, h1 = p.shape
    _, h2 = w.shape
    return pl.pallas_call(
        _relu_mm_body,
        grid=(n // bn,),
        in_specs=[pl.BlockSpec((1, bn, h1), lambda i: (0, i, 0)),
                  pl.BlockSpec((1, bn, h1), lambda i: (1, i, 0)),
                  pl.BlockSpec((h1, h2), lambda i: (0, 0))],
        out_specs=pl.BlockSpec((bn, h2), lambda i: (i, 0)),
        out_shape=jax.ShapeDtypeStruct((n, h2), jnp.float32),
    )(p, p, w)


def _decoder_body(pi0_ref, pi1_ref, pj0_ref, pj1_ref, z_ref, r_ref):
    zi = pi0_ref[0] + pi1_ref[0]
    zj = pj0_ref[0] + pj1_ref[0]
    z_ref[...] = zi
    r_ref[...] = lax.dot_general(
        zi, zj, dimension_numbers=(((1,), (1,)), ((), ())),
        preferred_element_type=jnp.float32)


def _decoder(p, n, bn):
    _, _, h = p.shape
    return pl.pallas_call(
        _decoder_body,
        grid=(n // bn,),
        in_specs=[pl.BlockSpec((1, bn, h), lambda i: (0, i, 0)),
                  pl.BlockSpec((1, bn, h), lambda i: (1, i, 0)),
                  pl.BlockSpec((1, n, h), lambda i: (0, 0, 0)),
                  pl.BlockSpec((1, n, h), lambda i: (1, 0, 0))],
        out_specs=[pl.BlockSpec((bn, h), lambda i: (i, 0)),
                   pl.BlockSpec((bn, n), lambda i: (i, 0))],
        out_shape=[jax.ShapeDtypeStruct((n, h), jnp.float32),
                   jax.ShapeDtypeStruct((n, n), jnp.float32)],
    )(p, p, p, p)


def kernel(x, edge_index, W1, W2):
    n = x.shape[0]
    h = _mm1(x, W1, 1000)
    p1 = _spmm_partials(edge_index, h)
    h2 = _relu_mm(p1, W2, n, 1000)
    p2 = _spmm_partials(edge_index, h2)
    z, recon = _decoder(p2, n, 400)
    return (z, jnp.reshape(recon, (-1,)))


# speedup vs baseline: 1.3126x; 1.0017x over previous
import functools

import jax
import jax.numpy as jnp
from jax import lax
from jax.experimental import pallas as pl
from jax.experimental.pallas import tpu as pltpu
from jax.experimental.pallas import tpu_sc as plsc

NC = 2
NS = 16
NW = NC * NS
CHUNK = 128
LANES = 16


GRP = 4


def _make_spmm(n, h, ngrp):
    zbuf_rows = 64
    zrows_tile = -(-(-(-(n + 1) // NS)) // zbuf_rows) * zbuf_rows
    acc_rows = NS * zrows_tile
    mesh = plsc.VectorSubcoreMesh(core_axis_name="c", subcore_axis_name="s")

    @functools.partial(
        pl.kernel,
        mesh=mesh,
        compiler_params=pltpu.CompilerParams(use_tc_tiling_on_sc=False),
        out_type=jax.ShapeDtypeStruct((NC, acc_rows, h), jnp.float32),
        scratch_types=[
            pltpu.VMEM((ngrp, GRP, CHUNK), jnp.int32),
            pltpu.VMEM((ngrp, GRP, CHUNK), jnp.int32),
            pltpu.VMEM((GRP, CHUNK, h), jnp.float32),
            pltpu.VMEM((GRP, CHUNK, h), jnp.float32),
            pltpu.VMEM((zbuf_rows, h), jnp.float32),
            pltpu.VMEM_SHARED((acc_rows, h), jnp.float32),
            pltpu.VMEM_SHARED((n, h), jnp.float32),
            pltpu.SemaphoreType.DMA,
            pltpu.SemaphoreType.DMA,
            pltpu.SemaphoreType.DMA,
            pltpu.SemaphoreType.DMA,
        ],
    )
    def spmm(src_hbm, dst_hbm, tbl_hbm, out_hbm, srcv, dstv, rows0, rows1,
             zbuf, acc, tbl_s, gsem0, gsem1, ssem0, ssem1):
        cid = lax.axis_index("c")
        sid = lax.axis_index("s")
        wid = cid * NS + sid

        zero = jnp.zeros((LANES,), jnp.float32)

        def zrow(r, carry):
            for j in range(h // LANES):
                zbuf[r, pl.ds(j * LANES, LANES)] = zero
            return carry

        lax.fori_loop(0, zbuf_rows, zrow, 0)
        for t in range(zrows_tile // zbuf_rows):
            pltpu.sync_copy(
                zbuf, acc.at[pl.ds(sid * zrows_tile + t * zbuf_rows, zbuf_rows)])
        plsc.subcore_barrier()

        pltpu.sync_copy(src_hbm.at[wid], srcv)
        pltpu.sync_copy(dst_hbm.at[wid], dstv)
        trows = n // NS
        tbase = sid * trows
        pltpu.sync_copy(tbl_hbm.at[pl.ds(tbase, trows)],
                        tbl_s.at[pl.ds(tbase, trows)])
        plsc.subcore_barrier()

        def fire_gathers(s, buf, sem):
            for k in range(GRP):
                pltpu.async_copy(tbl_s.at[srcv.at[s, k]], buf.at[k], sem)

        def drain_gathers(buf, sem):
            for k in range(GRP):
                pltpu.make_async_copy(
                    tbl_s.at[srcv.at[0, k]], buf.at[k], sem).wait()

        def fire_scatters(s, buf, sem):
            for k in range(GRP):
                pltpu.async_copy(buf.at[k], acc.at[dstv.at[s, k]], sem,
                                 add=True)

        def drain_scatters(s, buf, sem):
            for k in range(GRP):
                pltpu.make_async_copy(buf.at[k], acc.at[dstv.at[s, k]],
                                      sem).wait()

        banks = ((rows0, gsem0, ssem0), (rows1, gsem1, ssem1))

        def step(s, carry):
            def run(p):
                buf, gsem, ssem = banks[p]
                obuf, ogsem, ossem = banks[1 - p]

                @pl.when(s >= 1)
                def _():
                    drain_scatters(s - 1, obuf, ossem)

                @pl.when(s + 1 < ngrp)
                def _():
                    fire_gathers(s + 1, obuf, ogsem)

                drain_gathers(buf, gsem)
                fire_scatters(s, buf, ssem)

            @pl.when(lax.rem(s, 2) == 0)
            def _():
                run(0)

            @pl.when(lax.rem(s, 2) == 1)
            def _():
                run(1)

            return carry

        fire_gathers(0, rows0, gsem0)
        lax.fori_loop(0, ngrp, step, 0)
        lbuf, _, lssem = banks[(ngrp - 1) % 2]
        drain_scatters(ngrp - 1, lbuf, lssem)
        plsc.subcore_barrier()

        pltpu.sync_copy(acc.at[pl.ds(sid * zrows_tile, zrows_tile)],
                        out_hbm.at[cid, pl.ds(sid * zrows_tile, zrows_tile)])

    return spmm


def _prep_edges(edge_index, n):
    e = edge_index.shape[1]
    gsz = GRP * CHUNK
    per_w = -(-e // (NW * gsz)) * gsz
    e_pad = per_w * NW
    src = jnp.concatenate(
        [edge_index[0], jnp.zeros((e_pad - e,), jnp.int32)]).reshape(
            NW, per_w // gsz, GRP, CHUNK)
    dst = jnp.concatenate(
        [edge_index[1], jnp.full((e_pad - e,), n, jnp.int32)]).reshape(
            NW, per_w // gsz, GRP, CHUNK)
    return src, dst


def _spmm_partials(src, dst, table):
    n, h = table.shape
    return _make_spmm(n, h, src.shape[1])(src, dst, table)


def _mm1_body(x_ref, w_ref, o_ref):
    o_ref[...] = jnp.dot(x_ref[...], w_ref[...],
                         preferred_element_type=jnp.float32,
                         precision=lax.Precision.HIGHEST)


def _mm1(x, w, bn):
    n, d = x.shape
    _, h = w.shape
    return pl.pallas_call(
        _mm1_body,
        grid=(n // bn,),
        in_specs=[pl.BlockSpec((bn, d), lambda i: (i, 0)),
                  pl.BlockSpec((d, h), lambda i: (0, 0))],
        out_specs=pl.BlockSpec((bn, h), lambda i: (i, 0)),
        out_shape=jax.ShapeDtypeStruct((n, h), jnp.float32),
    )(x, w)


def _relu_mm_body(p0_ref, p1_ref, w_ref, o_ref):
    hblk = jnp.maximum(p0_ref[0] + p1_ref[0], 0.0)
    o_ref[...] = jnp.dot(hblk, w_ref[...],
                         preferred_element_type=jnp.float32,
                         precision=lax.Precision.HIGHEST)


def _relu_mm(p, w, n, bn):
    _, _, h1 = p.shape
    _, h2 = w.shape
    return pl.pallas_call(
        _relu_mm_body,
        grid=(n // bn,),
        in_specs=[pl.BlockSpec((1, bn, h1), lambda i: (0, i, 0)),
                  pl.BlockSpec((1, bn, h1), lambda i: (1, i, 0)),
                  pl.BlockSpec((h1, h2), lambda i: (0, 0))],
        out_specs=pl.BlockSpec((bn, h2), lambda i: (i, 0)),
        out_shape=jax.ShapeDtypeStruct((n, h2), jnp.float32),
    )(p, p, w)


def _decoder_body(pi0_ref, pi1_ref, pj0_ref, pj1_ref, z_ref, r_ref):
    zi = pi0_ref[0] + pi1_ref[0]
    zj = pj0_ref[0] + pj1_ref[0]
    z_ref[...] = zi
    r_ref[...] = lax.dot_general(
        zi, zj, dimension_numbers=(((1,), (1,)), ((), ())),
        preferred_element_type=jnp.float32)


def _decoder(p, n, bn):
    _, _, h = p.shape
    return pl.pallas_call(
        _decoder_body,
        grid=(n // bn,),
        in_specs=[pl.BlockSpec((1, bn, h), lambda i: (0, i, 0)),
                  pl.BlockSpec((1, bn, h), lambda i: (1, i, 0)),
                  pl.BlockSpec((1, n, h), lambda i: (0, 0, 0)),
                  pl.BlockSpec((1, n, h), lambda i: (1, 0, 0))],
        out_specs=[pl.BlockSpec((bn, h), lambda i: (i, 0)),
                   pl.BlockSpec((bn, n), lambda i: (i, 0))],
        out_shape=[jax.ShapeDtypeStruct((n, h), jnp.float32),
                   jax.ShapeDtypeStruct((n, n), jnp.float32)],
    )(p, p, p, p)


def kernel(x, edge_index, W1, W2):
    n = x.shape[0]
    src, dst = _prep_edges(edge_index, n)
    h = _mm1(x, W1, 1000)
    p1 = _spmm_partials(src, dst, h)
    h2 = _relu_mm(p1, W2, n, 1000)
    p2 = _spmm_partials(src, dst, h2)
    z, recon = _decoder(p2, n, 400)
    return (z, jnp.reshape(recon, (-1,)))


# speedup vs baseline: 1.3164x; 1.0029x over previous
import functools

import jax
import jax.numpy as jnp
from jax import lax
from jax.experimental import pallas as pl
from jax.experimental.pallas import tpu as pltpu
from jax.experimental.pallas import tpu_sc as plsc

NC = 2
NS = 16
NW = NC * NS
CHUNK = 128
LANES = 16


GRP = 5


def _make_spmm(n, h, ngrp):
    zbuf_rows = 32
    zrows_tile = -(-(-(-(n + 1) // NS)) // zbuf_rows) * zbuf_rows
    acc_rows = NS * zrows_tile
    mesh = plsc.VectorSubcoreMesh(core_axis_name="c", subcore_axis_name="s")

    @functools.partial(
        pl.kernel,
        mesh=mesh,
        compiler_params=pltpu.CompilerParams(use_tc_tiling_on_sc=False),
        out_type=jax.ShapeDtypeStruct((NC, acc_rows, h), jnp.float32),
        scratch_types=[
            pltpu.VMEM((ngrp, GRP, CHUNK), jnp.int32),
            pltpu.VMEM((ngrp, GRP, CHUNK), jnp.int32),
            pltpu.VMEM((GRP, CHUNK, h), jnp.float32),
            pltpu.VMEM((GRP, CHUNK, h), jnp.float32),
            pltpu.VMEM((zbuf_rows, h), jnp.float32),
            pltpu.VMEM_SHARED((acc_rows, h), jnp.float32),
            pltpu.VMEM_SHARED((n, h), jnp.float32),
            pltpu.SemaphoreType.DMA,
            pltpu.SemaphoreType.DMA,
            pltpu.SemaphoreType.DMA,
            pltpu.SemaphoreType.DMA,
        ],
    )
    def spmm(src_hbm, dst_hbm, tbl_hbm, out_hbm, srcv, dstv, rows0, rows1,
             zbuf, acc, tbl_s, gsem0, gsem1, ssem0, ssem1):
        cid = lax.axis_index("c")
        sid = lax.axis_index("s")
        wid = cid * NS + sid

        zero = jnp.zeros((LANES,), jnp.float32)

        def zrow(r, carry):
            for j in range(h // LANES):
                zbuf[r, pl.ds(j * LANES, LANES)] = zero
            return carry

        lax.fori_loop(0, zbuf_rows, zrow, 0)
        for t in range(zrows_tile // zbuf_rows):
            pltpu.sync_copy(
                zbuf, acc.at[pl.ds(sid * zrows_tile + t * zbuf_rows, zbuf_rows)])
        plsc.subcore_barrier()

        c_src = pltpu.async_copy(src_hbm.at[wid], srcv, gsem1)
        c_dst = pltpu.async_copy(dst_hbm.at[wid], dstv, gsem1)
        trows = (n // NS) // 8 * 8
        lrows = n - (NS - 1) * trows

        @pl.when(sid < NS - 1)
        def _():
            pltpu.async_copy(tbl_hbm.at[pl.ds(sid * trows, trows)],
                             tbl_s.at[pl.ds(sid * trows, trows)], ssem0)

        @pl.when(sid == NS - 1)
        def _():
            pltpu.async_copy(tbl_hbm.at[pl.ds((NS - 1) * trows, lrows)],
                             tbl_s.at[pl.ds((NS - 1) * trows, lrows)], ssem0)

        c_src.wait()
        c_dst.wait()

        @pl.when(sid < NS - 1)
        def _():
            pltpu.make_async_copy(tbl_hbm.at[pl.ds(sid * trows, trows)],
                                  tbl_s.at[pl.ds(sid * trows, trows)],
                                  ssem0).wait()

        @pl.when(sid == NS - 1)
        def _():
            pltpu.make_async_copy(tbl_hbm.at[pl.ds((NS - 1) * trows, lrows)],
                                  tbl_s.at[pl.ds((NS - 1) * trows, lrows)],
                                  ssem0).wait()

        plsc.subcore_barrier()

        def fire_gathers(s, buf, sem):
            for k in range(GRP):
                pltpu.async_copy(tbl_s.at[srcv.at[s, k]], buf.at[k], sem)

        def drain_gathers(buf, sem):
            for k in range(GRP):
                pltpu.make_async_copy(
                    tbl_s.at[srcv.at[0, k]], buf.at[k], sem).wait()

        def fire_scatters(s, buf, sem):
            for k in range(GRP):
                pltpu.async_copy(buf.at[k], acc.at[dstv.at[s, k]], sem,
                                 add=True)

        def drain_scatters(s, buf, sem):
            for k in range(GRP):
                pltpu.make_async_copy(buf.at[k], acc.at[dstv.at[s, k]],
                                      sem).wait()

        banks = ((rows0, gsem0, ssem0), (rows1, gsem1, ssem1))

        def step(s, carry):
            def run(p):
                buf, gsem, ssem = banks[p]
                obuf, ogsem, ossem = banks[1 - p]

                @pl.when(s >= 1)
                def _():
                    drain_scatters(s - 1, obuf, ossem)

                @pl.when(s + 1 < ngrp)
                def _():
                    fire_gathers(s + 1, obuf, ogsem)

                drain_gathers(buf, gsem)
                fire_scatters(s, buf, ssem)

            @pl.when(lax.rem(s, 2) == 0)
            def _():
                run(0)

            @pl.when(lax.rem(s, 2) == 1)
            def _():
                run(1)

            return carry

        fire_gathers(0, rows0, gsem0)
        lax.fori_loop(0, ngrp, step, 0)
        lbuf, _, lssem = banks[(ngrp - 1) % 2]
        drain_scatters(ngrp - 1, lbuf, lssem)
        plsc.subcore_barrier()

        pltpu.sync_copy(acc.at[pl.ds(sid * zrows_tile, zrows_tile)],
                        out_hbm.at[cid, pl.ds(sid * zrows_tile, zrows_tile)])

    return spmm


def _prep_edges(edge_index, n):
    e = edge_index.shape[1]
    gsz = GRP * CHUNK
    per_w = -(-e // (NW * gsz)) * gsz
    e_pad = per_w * NW
    src = jnp.concatenate(
        [edge_index[0], jnp.zeros((e_pad - e,), jnp.int32)]).reshape(
            NW, per_w // gsz, GRP, CHUNK)
    dst = jnp.concatenate(
        [edge_index[1], jnp.full((e_pad - e,), n, jnp.int32)]).reshape(
            NW, per_w // gsz, GRP, CHUNK)
    return src, dst


def _spmm_partials(src, dst, table):
    n, h = table.shape
    return _make_spmm(n, h, src.shape[1])(src, dst, table)


def _mm1_body(x_ref, w_ref, o_ref):
    o_ref[...] = jnp.dot(x_ref[...], w_ref[...],
                         preferred_element_type=jnp.float32,
                         precision=lax.Precision.HIGHEST)


def _mm1(x, w, bn):
    n, d = x.shape
    _, h = w.shape
    return pl.pallas_call(
        _mm1_body,
        grid=(n // bn,),
        in_specs=[pl.BlockSpec((bn, d), lambda i: (i, 0)),
                  pl.BlockSpec((d, h), lambda i: (0, 0))],
        out_specs=pl.BlockSpec((bn, h), lambda i: (i, 0)),
        out_shape=jax.ShapeDtypeStruct((n, h), jnp.float32),
    )(x, w)


def _relu_mm_body(p0_ref, p1_ref, w_ref, o_ref):
    hblk = jnp.maximum(p0_ref[0] + p1_ref[0], 0.0)
    o_ref[...] = jnp.dot(hblk, w_ref[...],
                         preferred_element_type=jnp.float32,
                         precision=lax.Precision.HIGHEST)


def _relu_mm(p, w, n, bn):
    _, _, h1 = p.shape
    _, h2 = w.shape
    return pl.pallas_call(
        _relu_mm_body,
        grid=(n // bn,),
        in_specs=[pl.BlockSpec((1, bn, h1), lambda i: (0, i, 0)),
                  pl.BlockSpec((1, bn, h1), lambda i: (1, i, 0)),
                  pl.BlockSpec((h1, h2), lambda i: (0, 0))],
        out_specs=pl.BlockSpec((bn, h2), lambda i: (i, 0)),
        out_shape=jax.ShapeDtypeStruct((n, h2), jnp.float32),
    )(p, p, w)


def _decoder_body(pi0_ref, pi1_ref, pj0_ref, pj1_ref, z_ref, r_ref):
    zi = pi0_ref[0] + pi1_ref[0]
    zj = pj0_ref[0] + pj1_ref[0]
    z_ref[...] = zi
    r_ref[...] = lax.dot_general(
        zi, zj, dimension_numbers=(((1,), (1,)), ((), ())),
        preferred_element_type=jnp.float32)


def _decoder(p, n, bn):
    _, _, h = p.shape
    return pl.pallas_call(
        _decoder_body,
        grid=(n // bn,),
        in_specs=[pl.BlockSpec((1, bn, h), lambda i: (0, i, 0)),
                  pl.BlockSpec((1, bn, h), lambda i: (1, i, 0)),
                  pl.BlockSpec((1, n, h), lambda i: (0, 0, 0)),
                  pl.BlockSpec((1, n, h), lambda i: (1, 0, 0))],
        out_specs=[pl.BlockSpec((bn, h), lambda i: (i, 0)),
                   pl.BlockSpec((bn, n), lambda i: (i, 0))],
        out_shape=[jax.ShapeDtypeStruct((n, h), jnp.float32),
                   jax.ShapeDtypeStruct((n, n), jnp.float32)],
    )(p, p, p, p)


def kernel(x, edge_index, W1, W2):
    n = x.shape[0]
    src, dst = _prep_edges(edge_index, n)
    h = _mm1(x, W1, 1000)
    p1 = _spmm_partials(src, dst, h)
    h2 = _relu_mm(p1, W2, n, 1000)
    p2 = _spmm_partials(src, dst, h2)
    z, recon = _decoder(p2, n, 400)
    return (z, jnp.reshape(recon, (-1,)))
